# baseline (speedup 1.0000x reference)
import functools

import jax
import jax.numpy as jnp
from jax import lax
from jax.experimental import pallas as pl
from jax.experimental.pallas import tpu as pltpu
from jax.experimental.pallas import tpu_sc as plsc

_B, _T, _D = 8, 2048, 512
_C, _V = 256, 8192
_M = _B * _T
_EPS = 1e-10

def _stats_body(x_ref, mean_ref, std_ref):
    xb = x_ref[...]
    m = jnp.mean(xb, axis=1, keepdims=True)
    c = xb - m
    var = jnp.sum(c * c, axis=1, keepdims=True) / (_T - 1)
    std = jnp.maximum(jnp.sqrt(var), _EPS)
    mean_ref[...] = m
    std_ref[...] = std


def _stats(x):
    return pl.pallas_call(
        _stats_body,
        grid=(_B,),
        in_specs=[pl.BlockSpec((1, _T, _D), lambda b: (b, 0, 0))],
        out_specs=[
            pl.BlockSpec((1, 1, _D), lambda b: (b, 0, 0)),
            pl.BlockSpec((1, 1, _D), lambda b: (b, 0, 0)),
        ],
        out_shape=[
            jax.ShapeDtypeStruct((_B, 1, _D), jnp.float32),
            jax.ShapeDtypeStruct((_B, 1, _D), jnp.float32),
        ],
    )(x)


_CB_BLK = 1024


def _cbn_body(cb_ref, out_ref):
    blk = cb_ref[...]
    nrm = jnp.sqrt(jnp.sum(blk * blk, axis=1, keepdims=True))
    out_ref[...] = blk / nrm


def _cbn(cb):
    return pl.pallas_call(
        _cbn_body,
        grid=(_V // _CB_BLK,),
        in_specs=[pl.BlockSpec((_CB_BLK, _C), lambda i: (i, 0))],
        out_specs=pl.BlockSpec((_CB_BLK, _C), lambda i: (i, 0)),
        out_shape=jax.ShapeDtypeStruct((_V, _C), jnp.float32),
    )(cb)


_BM = 256
_BV = 128


def _argmax_body(x_ref, mean_ref, std_ref, p_ref, cbn_ref, t_ref):
    gm = jnp.mean(mean_ref[...], axis=0)
    gs = jnp.mean(std_ref[...], axis=0)
    xn = (x_ref[...] - gm) / gs
    xp = lax.dot_general(
        xn, p_ref[...], (((1,), (0,)), ((), ())),
        preferred_element_type=jnp.float32)
    nrm = jnp.sqrt(jnp.sum(xp * xp, axis=1, keepdims=True))
    xpn = xp / nrm

    sub_iota = lax.broadcasted_iota(jnp.int32, (8, _BM), 0)

    def sub_step(v, carry):
        m8, i8 = carry
        cbl = cbn_ref[pl.ds(v * _BV, _BV), :]
        simt = lax.dot_general(
            cbl, xpn, (((1,), (1,)), ((), ())),
            preferred_element_type=jnp.float32)
        for s in range(_BV // 8):
            blk = lax.slice(simt, (s * 8, 0), (s * 8 + 8, _BM))
            base = jnp.full((8, _BM), v * _BV + s * 8, jnp.int32)
            gt = blk > m8
            m8 = jnp.where(gt, blk, m8)
            i8 = jnp.where(gt, base, i8)
        return m8, i8

    init = (jnp.full((8, _BM), -jnp.inf, jnp.float32),
            jnp.zeros((8, _BM), jnp.int32))
    m8, i8 = lax.fori_loop(0, _V // _BV, sub_step, init, unroll=2)
    i8 = i8 + sub_iota
    m4, i4 = m8.reshape(2, 4, _BM), i8.reshape(2, 4, _BM)
    for _ in range(3):
        lo_m, hi_m = m4[0], m4[1]
        lo_i, hi_i = i4[0], i4[1]
        take_hi = (hi_m > lo_m) | ((hi_m == lo_m) & (hi_i < lo_i))
        mm = jnp.where(take_hi, hi_m, lo_m)
        ii = jnp.where(take_hi, hi_i, lo_i)
        k = mm.shape[0]
        if k > 1:
            m4, i4 = mm.reshape(2, k // 2, _BM), ii.reshape(2, k // 2, _BM)
        else:
            m4, i4 = mm, ii
    t_ref[...] = ii.reshape(1, 1, _BM)


def _targets(x2d, means, stds, P, cbn):
    nblk = _M // _BM
    t3 = pl.pallas_call(
        _argmax_body,
        grid=(nblk,),
        in_specs=[
            pl.BlockSpec((_BM, _D), lambda i: (i, 0)),
            pl.BlockSpec((_B, 1, _D), lambda i: (0, 0, 0)),
            pl.BlockSpec((_B, 1, _D), lambda i: (0, 0, 0)),
            pl.BlockSpec((_D, _C), lambda i: (0, 0)),
            pl.BlockSpec((_V, _C), lambda i: (0, 0)),
        ],
        out_specs=pl.BlockSpec((1, 1, _BM), lambda i: (i, 0, 0)),
        out_shape=jax.ShapeDtypeStruct((nblk, 1, _BM), jnp.int32),
        compiler_params=pltpu.CompilerParams(
            dimension_semantics=("arbitrary",)),
    )(x2d, means, stds, P, cbn)
    return t3.reshape(_M)


_NC, _NS = 2, 16
_NW = _NC * _NS
_BPW = _M // _NW
_CH = 128


@functools.cache
def _gather_rows_kernel():
    @functools.partial(
        pl.kernel,
        mesh=plsc.VectorSubcoreMesh(core_axis_name="c", subcore_axis_name="s"),
        out_type=jax.ShapeDtypeStruct((_M, _C), jnp.float32),
        scratch_types=[
            pltpu.VMEM((_CH,), jnp.int32),
            pltpu.VMEM((_CH, _C), jnp.float32),
            pltpu.SemaphoreType.DMA,
        ],
    )
    def _gather_rows(table_hbm, idx_hbm, out_hbm, idx_v, rows_v, sem):
        wid = lax.axis_index("s") * _NC + lax.axis_index("c")
        base = wid * _BPW
        for ch in range(_BPW // _CH):
            start = base + ch * _CH
            pltpu.sync_copy(idx_hbm.at[pl.ds(start, _CH)], idx_v)
            pltpu.async_copy(table_hbm.at[idx_v], rows_v, sem).wait()
            pltpu.sync_copy(rows_v, out_hbm.at[pl.ds(start, _CH)])

    return _gather_rows


def kernel(x, lengths, P, cb):
    del lengths
    means, stds = _stats(x)
    cbn = _cbn(cb)
    tflat = _targets(x.reshape(_M, _D), means, stds, P, cbn)
    quant = _gather_rows_kernel()(cbn, tflat)
    return quant.reshape(_B, _T, _C), tflat.reshape(_B, _T)


# speedup vs baseline: 4.0138x; 4.0138x over previous
import functools

import jax
import jax.numpy as jnp
from jax import lax
from jax.experimental import pallas as pl
from jax.experimental.pallas import tpu as pltpu
from jax.experimental.pallas import tpu_sc as plsc

_B, _T, _D = 8, 2048, 512
_C, _V = 256, 8192
_M = _B * _T
_EPS = 1e-10

def _stats_body(x_ref, mean_ref, std_ref):
    xb = x_ref[...]
    m = jnp.mean(xb, axis=1, keepdims=True)
    c = xb - m
    var = jnp.sum(c * c, axis=1, keepdims=True) / (_T - 1)
    std = jnp.maximum(jnp.sqrt(var), _EPS)
    mean_ref[...] = m
    std_ref[...] = std


def _stats(x):
    return pl.pallas_call(
        _stats_body,
        grid=(_B,),
        in_specs=[pl.BlockSpec((1, _T, _D), lambda b: (b, 0, 0))],
        out_specs=[
            pl.BlockSpec((1, 1, _D), lambda b: (b, 0, 0)),
            pl.BlockSpec((1, 1, _D), lambda b: (b, 0, 0)),
        ],
        out_shape=[
            jax.ShapeDtypeStruct((_B, 1, _D), jnp.float32),
            jax.ShapeDtypeStruct((_B, 1, _D), jnp.float32),
        ],
    )(x)


_CB_BLK = 1024


def _cbn_body(cb_ref, out_ref):
    blk = cb_ref[...]
    nrm = jnp.sqrt(jnp.sum(blk * blk, axis=1, keepdims=True))
    out_ref[...] = blk / nrm


def _cbn(cb):
    return pl.pallas_call(
        _cbn_body,
        grid=(_V // _CB_BLK,),
        in_specs=[pl.BlockSpec((_CB_BLK, _C), lambda i: (i, 0))],
        out_specs=pl.BlockSpec((_CB_BLK, _C), lambda i: (i, 0)),
        out_shape=jax.ShapeDtypeStruct((_V, _C), jnp.float32),
    )(cb)


_BM = 512
_BV = 1024


def _argmax_body(x_ref, mean_ref, std_ref, p_ref, cbn_ref, t_ref,
                 scr_a, scr_b):
    gm = jnp.mean(mean_ref[...], axis=0)
    gs = jnp.mean(std_ref[...], axis=0)
    xn = (x_ref[...] - gm) / gs
    xp = lax.dot_general(
        xn, p_ref[...], (((1,), (0,)), ((), ())),
        preferred_element_type=jnp.float32)
    nrm = jnp.sqrt(jnp.sum(xp * xp, axis=1, keepdims=True))
    xpn = xp / nrm

    sub_iota = lax.broadcasted_iota(jnp.int32, (8, _BM), 0)

    def do_dot(v, scr):
        cbl = cbn_ref[pl.ds(v * _BV, _BV), :]
        scr[...] = lax.dot_general(
            cbl, xpn, (((1,), (1,)), ((), ())),
            preferred_element_type=jnp.float32)

    def do_scan(v, scr, carry):
        m8, i8 = carry
        for s in range(_BV // 8):
            blk = scr[s * 8:(s + 1) * 8, :]
            base = jnp.full((8, _BM), v * _BV + s * 8, jnp.int32)
            gt = blk > m8
            m8 = jnp.where(gt, blk, m8)
            i8 = jnp.where(gt, base, i8)
        return m8, i8

    nchunks = _V // _BV
    bufs = (scr_a, scr_b)
    carry = (jnp.full((8, _BM), -jnp.inf, jnp.float32),
             jnp.zeros((8, _BM), jnp.int32))
    do_dot(0, bufs[0])
    for v in range(nchunks):
        if v + 1 < nchunks:
            do_dot(v + 1, bufs[(v + 1) % 2])
        carry = do_scan(v, bufs[v % 2], carry)
    m8, i8 = carry
    i8 = i8 + sub_iota
    m4, i4 = m8.reshape(2, 4, _BM), i8.reshape(2, 4, _BM)
    for _ in range(3):
        lo_m, hi_m = m4[0], m4[1]
        lo_i, hi_i = i4[0], i4[1]
        take_hi = (hi_m > lo_m) | ((hi_m == lo_m) & (hi_i < lo_i))
        mm = jnp.where(take_hi, hi_m, lo_m)
        ii = jnp.where(take_hi, hi_i, lo_i)
        k = mm.shape[0]
        if k > 1:
            m4, i4 = mm.reshape(2, k // 2, _BM), ii.reshape(2, k // 2, _BM)
        else:
            m4, i4 = mm, ii
    t_ref[...] = ii.reshape(1, 1, _BM)


def _targets(x2d, means, stds, P, cbn):
    nblk = _M // _BM
    t3 = pl.pallas_call(
        _argmax_body,
        grid=(nblk,),
        in_specs=[
            pl.BlockSpec((_BM, _D), lambda i: (i, 0)),
            pl.BlockSpec((_B, 1, _D), lambda i: (0, 0, 0)),
            pl.BlockSpec((_B, 1, _D), lambda i: (0, 0, 0)),
            pl.BlockSpec((_D, _C), lambda i: (0, 0)),
            pl.BlockSpec((_V, _C), lambda i: (0, 0)),
        ],
        out_specs=pl.BlockSpec((1, 1, _BM), lambda i: (i, 0, 0)),
        out_shape=jax.ShapeDtypeStruct((nblk, 1, _BM), jnp.int32),
        scratch_shapes=[pltpu.VMEM((_BV, _BM), jnp.float32),
                        pltpu.VMEM((_BV, _BM), jnp.float32)],
        compiler_params=pltpu.CompilerParams(
            dimension_semantics=("arbitrary",)),
    )(x2d, means, stds, P, cbn)
    return t3.reshape(_M)


_NC, _NS = 2, 16
_NW = _NC * _NS
_BPW = _M // _NW
_CH = 128


@functools.cache
def _gather_rows_kernel():
    @functools.partial(
        pl.kernel,
        mesh=plsc.VectorSubcoreMesh(core_axis_name="c", subcore_axis_name="s"),
        out_type=jax.ShapeDtypeStruct((_M, _C), jnp.float32),
        scratch_types=[
            pltpu.VMEM((_CH,), jnp.int32),
            pltpu.VMEM((_CH, _C), jnp.float32),
            pltpu.SemaphoreType.DMA,
        ],
    )
    def _gather_rows(table_hbm, idx_hbm, out_hbm, idx_v, rows_v, sem):
        wid = lax.axis_index("s") * _NC + lax.axis_index("c")
        base = wid * _BPW
        for ch in range(_BPW // _CH):
            start = base + ch * _CH
            pltpu.sync_copy(idx_hbm.at[pl.ds(start, _CH)], idx_v)
            pltpu.async_copy(table_hbm.at[idx_v], rows_v, sem).wait()
            pltpu.sync_copy(rows_v, out_hbm.at[pl.ds(start, _CH)])

    return _gather_rows


def kernel(x, lengths, P, cb):
    del lengths
    means, stds = _stats(x)
    cbn = _cbn(cb)
    tflat = _targets(x.reshape(_M, _D), means, stds, P, cbn)
    quant = _gather_rows_kernel()(cbn, tflat)
    return quant.reshape(_B, _T, _C), tflat.reshape(_B, _T)


# speedup vs baseline: 4.2368x; 1.0556x over previous
import functools

import jax
import jax.numpy as jnp
from jax import lax
from jax.experimental import pallas as pl
from jax.experimental.pallas import tpu as pltpu
from jax.experimental.pallas import tpu_sc as plsc

_B, _T, _D = 8, 2048, 512
_C, _V = 256, 8192
_M = _B * _T
_EPS = 1e-10

def _stats_body(x_ref, mean_ref, std_ref):
    xb = x_ref[...]
    m = jnp.mean(xb, axis=1, keepdims=True)
    c = xb - m
    var = jnp.sum(c * c, axis=1, keepdims=True) / (_T - 1)
    std = jnp.maximum(jnp.sqrt(var), _EPS)
    mean_ref[...] = m
    std_ref[...] = std


def _stats(x):
    return pl.pallas_call(
        _stats_body,
        grid=(_B,),
        in_specs=[pl.BlockSpec((1, _T, _D), lambda b: (b, 0, 0))],
        out_specs=[
            pl.BlockSpec((1, 1, _D), lambda b: (b, 0, 0)),
            pl.BlockSpec((1, 1, _D), lambda b: (b, 0, 0)),
        ],
        out_shape=[
            jax.ShapeDtypeStruct((_B, 1, _D), jnp.float32),
            jax.ShapeDtypeStruct((_B, 1, _D), jnp.float32),
        ],
    )(x)


_CB_BLK = 1024


def _cbn_body(cb_ref, out_ref):
    blk = cb_ref[...]
    nrm = jnp.sqrt(jnp.sum(blk * blk, axis=1, keepdims=True))
    out_ref[...] = blk / nrm


def _cbn(cb):
    return pl.pallas_call(
        _cbn_body,
        grid=(_V // _CB_BLK,),
        in_specs=[pl.BlockSpec((_CB_BLK, _C), lambda i: (i, 0))],
        out_specs=pl.BlockSpec((_CB_BLK, _C), lambda i: (i, 0)),
        out_shape=jax.ShapeDtypeStruct((_V, _C), jnp.float32),
    )(cb)


_BM = 1024
_BV = 1024


def _argmax_body(x_ref, mean_ref, std_ref, p_ref, cbn_ref, t_ref,
                 scr_a, scr_b):
    gm = jnp.mean(mean_ref[...], axis=0)
    gs = jnp.mean(std_ref[...], axis=0)
    xn = (x_ref[...] - gm) / gs
    xp = lax.dot_general(
        xn, p_ref[...], (((1,), (0,)), ((), ())),
        preferred_element_type=jnp.float32)
    nrm = jnp.sqrt(jnp.sum(xp * xp, axis=1, keepdims=True))
    xpn = xp / nrm

    sub_iota = lax.broadcasted_iota(jnp.int32, (8, _BM), 0)

    def do_dot(v, scr):
        cbl = cbn_ref[pl.ds(v * _BV, _BV), :]
        scr[...] = lax.dot_general(
            cbl, xpn, (((1,), (1,)), ((), ())),
            preferred_element_type=jnp.float32)

    def do_scan(v, scr, carry):
        m8, i8 = carry
        for s in range(_BV // 8):
            blk = scr[s * 8:(s + 1) * 8, :]
            base = jnp.full((8, _BM), v * _BV + s * 8, jnp.int32)
            gt = blk > m8
            m8 = jnp.maximum(m8, blk)
            i8 = jnp.where(gt, base, i8)
        return m8, i8

    nchunks = _V // _BV
    bufs = (scr_a, scr_b)
    carry = (jnp.full((8, _BM), -jnp.inf, jnp.float32),
             jnp.zeros((8, _BM), jnp.int32))
    do_dot(0, bufs[0])
    for v in range(nchunks):
        if v + 1 < nchunks:
            do_dot(v + 1, bufs[(v + 1) % 2])
        carry = do_scan(v, bufs[v % 2], carry)
    m8, i8 = carry
    i8 = i8 + sub_iota
    m4, i4 = m8.reshape(2, 4, _BM), i8.reshape(2, 4, _BM)
    for _ in range(3):
        lo_m, hi_m = m4[0], m4[1]
        lo_i, hi_i = i4[0], i4[1]
        take_hi = (hi_m > lo_m) | ((hi_m == lo_m) & (hi_i < lo_i))
        mm = jnp.where(take_hi, hi_m, lo_m)
        ii = jnp.where(take_hi, hi_i, lo_i)
        k = mm.shape[0]
        if k > 1:
            m4, i4 = mm.reshape(2, k // 2, _BM), ii.reshape(2, k // 2, _BM)
        else:
            m4, i4 = mm, ii
    t_ref[...] = ii.reshape(1, 1, _BM)


def _targets(x2d, means, stds, P, cbn):
    nblk = _M // _BM
    t3 = pl.pallas_call(
        _argmax_body,
        grid=(nblk,),
        in_specs=[
            pl.BlockSpec((_BM, _D), lambda i: (i, 0)),
            pl.BlockSpec((_B, 1, _D), lambda i: (0, 0, 0)),
            pl.BlockSpec((_B, 1, _D), lambda i: (0, 0, 0)),
            pl.BlockSpec((_D, _C), lambda i: (0, 0)),
            pl.BlockSpec((_V, _C), lambda i: (0, 0)),
        ],
        out_specs=pl.BlockSpec((1, 1, _BM), lambda i: (i, 0, 0)),
        out_shape=jax.ShapeDtypeStruct((nblk, 1, _BM), jnp.int32),
        scratch_shapes=[pltpu.VMEM((_BV, _BM), jnp.float32),
                        pltpu.VMEM((_BV, _BM), jnp.float32)],
        compiler_params=pltpu.CompilerParams(
            dimension_semantics=("arbitrary",)),
    )(x2d, means, stds, P, cbn)
    return t3.reshape(_M)


_NC, _NS = 2, 16
_NW = _NC * _NS
_BPW = _M // _NW
_CH = 128


@functools.cache
def _gather_rows_kernel():
    @functools.partial(
        pl.kernel,
        mesh=plsc.VectorSubcoreMesh(core_axis_name="c", subcore_axis_name="s"),
        out_type=jax.ShapeDtypeStruct((_M, _C), jnp.float32),
        scratch_types=[
            pltpu.VMEM((_CH,), jnp.int32),
            pltpu.VMEM((_CH, _C), jnp.float32),
            pltpu.SemaphoreType.DMA,
        ],
    )
    def _gather_rows(table_hbm, idx_hbm, out_hbm, idx_v, rows_v, sem):
        wid = lax.axis_index("s") * _NC + lax.axis_index("c")
        base = wid * _BPW
        for ch in range(_BPW // _CH):
            start = base + ch * _CH
            pltpu.sync_copy(idx_hbm.at[pl.ds(start, _CH)], idx_v)
            pltpu.async_copy(table_hbm.at[idx_v], rows_v, sem).wait()
            pltpu.sync_copy(rows_v, out_hbm.at[pl.ds(start, _CH)])

    return _gather_rows


def kernel(x, lengths, P, cb):
    del lengths
    means, stds = _stats(x)
    cbn = _cbn(cb)
    tflat = _targets(x.reshape(_M, _D), means, stds, P, cbn)
    quant = _gather_rows_kernel()(cbn, tflat)
    return quant.reshape(_B, _T, _C), tflat.reshape(_B, _T)
